# Initial kernel scaffold; baseline (speedup 1.0000x reference)
import jax
import jax.numpy as jnp
from jax.experimental import pallas as pl


def kernel(x, rows, cols, vals, W, bias):
    raise NotImplementedError("write your pallas kernel here")


# speedup vs baseline: 10.5735x; 10.5735x over previous
import dataclasses
import functools

import jax
import jax.numpy as jnp
from jax import lax
from jax.experimental import pallas as pl
from jax.experimental.pallas import tpu as pltpu
from jax.experimental.pallas import tpu_sc as plsc

_B = 32
_N_IN = 16384
_N_OUT = 16384
_NNZ = 262144
_C = 64

_NC = 2
_NS = 16
_EPT = _NNZ // _NS
_K = 128
_NCH = _EPT // _K
_RPT = _N_OUT // _NS
_BPC = _B // _NC


def _mm_body(x_ref, w_ref, y_ref):
    y_ref[...] = lax.dot_general(
        x_ref[...], w_ref[...], (((1,), (0,)), ((), ())),
        preferred_element_type=jnp.float32,
        precision=lax.Precision.HIGHEST)


def _project(xf, W):
    blk = 2048
    return pl.pallas_call(
        _mm_body,
        grid=(xf.shape[0] // blk,),
        in_specs=[pl.BlockSpec((blk, _C), lambda i: (i, 0)),
                  pl.BlockSpec((_C, _C), lambda i: (0, 0))],
        out_specs=pl.BlockSpec((blk, _C), lambda i: (i, 0)),
        out_shape=jax.ShapeDtypeStruct((xf.shape[0], _C), jnp.float32),
    )(xf, W)


def _sc_aggregate(y_flat, rows3, cols, vals, bias):
    mesh = plsc.VectorSubcoreMesh(core_axis_name="c", subcore_axis_name="s")
    cp = pltpu.CompilerParams(use_tc_tiling_on_sc=False)
    if "needs_layout_passes" in pltpu.CompilerParams.__dataclass_fields__:
        cp = dataclasses.replace(cp, needs_layout_passes=False)

    @functools.partial(
        pl.kernel,
        compiler_params=cp,
        out_type=jax.ShapeDtypeStruct((_B, _N_OUT, _C), jnp.float32),
        mesh=mesh,
        scratch_types=[
            pltpu.VMEM((_EPT,), jnp.int32),
            pltpu.VMEM((_NCH, _K), jnp.int32),
            pltpu.VMEM((_EPT,), jnp.float32),
            pltpu.VMEM((_K, _C), jnp.float32),
            pltpu.VMEM_SHARED((_N_OUT, _C), jnp.float32),
            pltpu.SemaphoreType.DMA,
        ],
    )
    def k(y_hbm, rows_hbm, cols_hbm, vals_hbm, bias_hbm, out_hbm,
          colsb_v, rows_v, vals_v, g_v, acc, sem):
        cid = lax.axis_index("c")
        sid = lax.axis_index("s")
        ebase = sid * _EPT
        rbase = sid * _RPT

        pltpu.sync_copy(rows_hbm.at[sid], rows_v)
        pltpu.sync_copy(cols_hbm.at[pl.ds(ebase, _EPT)], colsb_v)
        pltpu.sync_copy(vals_hbm.at[pl.ds(ebase, _EPT)], vals_v)

        off0 = cid * (_BPC * _N_IN)

        @pl.loop(0, _EPT, step=16)
        def _(i):
            colsb_v[pl.ds(i, 16)] = colsb_v[pl.ds(i, 16)] + off0

        @pl.loop(0, _BPC)
        def _(bl):
            b = cid * _BPC + bl
            pltpu.sync_copy(bias_hbm.at[pl.ds(rbase, _RPT)],
                            acc.at[pl.ds(rbase, _RPT)])
            plsc.subcore_barrier()

            @pl.loop(0, _NCH)
            def _(j):
                pltpu.async_copy(
                    y_hbm.at[colsb_v.at[pl.ds(j * _K, _K)]], g_v, sem
                ).wait()

                @pl.loop(0, _K, step=16)
                def _(k0):
                    for kk in range(16):
                        idxv = jnp.full((16,), j * _K + k0 + kk, jnp.int32)
                        v = plsc.load_gather(vals_v, [idxv])
                        for c in range(_C // 16):
                            sl = pl.ds(c * 16, 16)
                            g_v[k0 + kk, sl] = g_v[k0 + kk, sl] * v

                pltpu.sync_copy(g_v, acc.at[rows_v.at[j]], add=True)

            plsc.subcore_barrier()
            pltpu.sync_copy(acc.at[pl.ds(rbase, _RPT)],
                            out_hbm.at[b, pl.ds(rbase, _RPT)])

            @pl.loop(0, _EPT, step=16)
            def _(i):
                colsb_v[pl.ds(i, 16)] = colsb_v[pl.ds(i, 16)] + _N_IN

    return k(y_flat, rows3, cols, vals, bias)


def kernel(x, rows, cols, vals, W, bias):
    B, n_in, C = x.shape
    assert (B, n_in, C) == (_B, _N_IN, _C) and rows.shape == (_NNZ,)
    y = _project(x.reshape(B * n_in, C), W)
    rows3 = rows.reshape(_NS, _NCH, _K)
    return _sc_aggregate(y, rows3, cols, vals, bias)


# speedup vs baseline: 22.7732x; 2.1538x over previous
import dataclasses
import functools

import jax
import jax.numpy as jnp
from jax import lax
from jax.experimental import pallas as pl
from jax.experimental.pallas import tpu as pltpu
from jax.experimental.pallas import tpu_sc as plsc

_B = 32
_N_IN = 16384
_N_OUT = 16384
_NNZ = 262144
_C = 64

_NC = 2
_NS = 16
_EPT = _NNZ // _NS
_K = 128
_NCH = _EPT // _K
_RPT = _N_OUT // _NS
_BPC = _B // _NC


def _mm_body(x_ref, w_ref, y_ref):
    y_ref[...] = lax.dot_general(
        x_ref[...], w_ref[...], (((1,), (0,)), ((), ())),
        preferred_element_type=jnp.float32,
        precision=lax.Precision.HIGHEST).astype(jnp.bfloat16)


def _project(xf, W):
    blk = 2048
    return pl.pallas_call(
        _mm_body,
        grid=(xf.shape[0] // blk,),
        in_specs=[pl.BlockSpec((blk, _C), lambda i: (i, 0)),
                  pl.BlockSpec((_C, _C), lambda i: (0, 0))],
        out_specs=pl.BlockSpec((blk, _C), lambda i: (i, 0)),
        out_shape=jax.ShapeDtypeStruct((xf.shape[0], _C), jnp.bfloat16),
    )(xf, W)


def _interleave_perm():
    perm = [0] * _C
    for c in range(_C // 32):
        for i in range(16):
            perm[32 * c + 2 * i] = 32 * c + i
            perm[32 * c + 2 * i + 1] = 32 * c + 16 + i
    return jnp.array(perm, jnp.int32)


def _sc_aggregate(y_flat, rows3, cols, vals, bias):
    mesh = plsc.VectorSubcoreMesh(core_axis_name="c", subcore_axis_name="s")
    cp = pltpu.CompilerParams(use_tc_tiling_on_sc=False)
    if "needs_layout_passes" in pltpu.CompilerParams.__dataclass_fields__:
        cp = dataclasses.replace(cp, needs_layout_passes=False)

    @functools.partial(
        pl.kernel,
        compiler_params=cp,
        out_type=jax.ShapeDtypeStruct((_B, _N_OUT, _C), jnp.float32),
        mesh=mesh,
        scratch_types=[
            pltpu.VMEM((_EPT,), jnp.int32),
            pltpu.VMEM((_NCH, _K), jnp.int32),
            pltpu.VMEM((_EPT,), jnp.float32),
            pltpu.VMEM((_K, _C), jnp.bfloat16),
            pltpu.VMEM((_K, _C), jnp.bfloat16),
            pltpu.VMEM((_K, _C), jnp.float32),
            pltpu.VMEM_SHARED((_N_OUT, _C), jnp.float32),
            pltpu.SemaphoreType.DMA,
            pltpu.SemaphoreType.DMA,
        ],
    )
    def k(y_hbm, rows_hbm, cols_hbm, vals_hbm, bias_hbm, out_hbm,
          colsb_v, rows_v, vals_v, g0, g1, sbuf, acc, gs0, gs1):
        gbufs = (g0, g1)
        gsems = (gs0, gs1)
        cid = lax.axis_index("c")
        sid = lax.axis_index("s")
        ebase = sid * _EPT
        rbase = sid * _RPT

        def gather_start(b, j, u):
            pltpu.async_copy(
                y_hbm.at[b].at[colsb_v.at[pl.ds(j * _K, _K)]],
                gbufs[u], gsems[u])

        def gather_wait(b, j, u):
            pltpu.make_async_copy(
                y_hbm.at[b].at[colsb_v.at[pl.ds(j * _K, _K)]],
                gbufs[u], gsems[u]
            ).wait()

        def scale(g, s, j):
            @plsc.parallel_loop(0, _K, step=16, unroll=2)
            def _(k0):
                vv = vals_v[pl.ds(j * _K + k0, 16)]
                dnums = lax.GatherDimensionNumbers(
                    offset_dims=(), collapsed_slice_dims=(0,),
                    start_index_map=(0,))
                bcasts = [
                    lax.gather(
                        vv, jnp.full((16, 1), kk, jnp.int32), dnums,
                        slice_sizes=(1,),
                        mode=lax.GatherScatterMode.PROMISE_IN_BOUNDS)
                    for kk in range(16)
                ]
                ncs = _C // 32
                for kk in range(16):
                    packs = [g[k0 + kk, pl.ds(c * 32, 32)] for c in range(ncs)]
                    halves = [
                        plsc.unpack(p, format=plsc.PackFormat.INTERLEAVED,
                                    preferred_element_type=jnp.float32)
                        for p in packs
                    ]
                    prods = [(a * bcasts[kk], b * bcasts[kk])
                             for (a, b) in halves]
                    for c in range(ncs):
                        s[k0 + kk, pl.ds(c * 32, 16)] = prods[c][0]
                        s[k0 + kk, pl.ds(c * 32 + 16, 16)] = prods[c][1]

        pltpu.sync_copy(rows_hbm.at[sid], rows_v)
        pltpu.sync_copy(cols_hbm.at[pl.ds(ebase, _EPT)], colsb_v)
        pltpu.sync_copy(vals_hbm.at[pl.ds(ebase, _EPT)], vals_v)

        @pl.loop(0, _BPC)
        def _(bl):
            b = cid * _BPC + bl
            pltpu.sync_copy(bias_hbm.at[pl.ds(rbase, _RPT)],
                            acc.at[pl.ds(rbase, _RPT)])
            plsc.subcore_barrier()

            gather_start(b, 0, 0)
            gather_start(b, 1, 1)

            @pl.loop(0, _NCH, step=2)
            def _(j0):
                for u in range(2):
                    j = j0 + u
                    gather_wait(b, j, u)
                    scale(gbufs[u], sbuf, j)
                    jn = j + 2

                    @pl.when(jn < _NCH)
                    def _():
                        gather_start(b, jn, u)

                    pltpu.sync_copy(sbuf, acc.at[rows_v.at[j]], add=True)

            plsc.subcore_barrier()
            pltpu.sync_copy(acc.at[pl.ds(rbase, _RPT)],
                            out_hbm.at[b, pl.ds(rbase, _RPT)])

    return k(y_flat, rows3, cols, vals, bias)


def kernel(x, rows, cols, vals, W, bias):
    B, n_in, C = x.shape
    assert (B, n_in, C) == (_B, _N_IN, _C) and rows.shape == (_NNZ,)
    y = _project(x.reshape(B * n_in, C), W[:, _interleave_perm()]
                 ).reshape(B, n_in, C)
    rows3 = rows.reshape(_NS, _NCH, _K)
    return _sc_aggregate(y, rows3, cols, vals, bias)


# speedup vs baseline: 24.3819x; 1.0706x over previous
import dataclasses
import functools

import jax
import jax.numpy as jnp
from jax import lax
from jax.experimental import pallas as pl
from jax.experimental.pallas import tpu as pltpu
from jax.experimental.pallas import tpu_sc as plsc

_B = 32
_N_IN = 16384
_N_OUT = 16384
_NNZ = 262144
_C = 64

_NC = 2
_NS = 16
_EPT = _NNZ // _NS
_K = 128
_NCH = _EPT // _K
_RPT = _N_OUT // _NS
_BPC = _B // _NC


def _mm_body(x_ref, w_ref, y_ref):
    y_ref[...] = lax.dot_general(
        x_ref[...], w_ref[...], (((1,), (0,)), ((), ())),
        preferred_element_type=jnp.float32,
        precision=lax.Precision.HIGHEST).astype(jnp.bfloat16)


def _project(xf, W):
    blk = 2048
    return pl.pallas_call(
        _mm_body,
        grid=(xf.shape[0] // blk,),
        in_specs=[pl.BlockSpec((blk, _C), lambda i: (i, 0)),
                  pl.BlockSpec((_C, _C), lambda i: (0, 0))],
        out_specs=pl.BlockSpec((blk, _C), lambda i: (i, 0)),
        out_shape=jax.ShapeDtypeStruct((xf.shape[0], _C), jnp.bfloat16),
    )(xf, W)


def _interleave_perm():
    perm = [0] * _C
    for c in range(_C // 32):
        for i in range(16):
            perm[32 * c + 2 * i] = 32 * c + i
            perm[32 * c + 2 * i + 1] = 32 * c + 16 + i
    return jnp.array(perm, jnp.int32)


def _sc_aggregate(y_flat, rows3, cols, vals, bias):
    mesh = plsc.VectorSubcoreMesh(core_axis_name="c", subcore_axis_name="s")
    cp = pltpu.CompilerParams(use_tc_tiling_on_sc=False)
    if "needs_layout_passes" in pltpu.CompilerParams.__dataclass_fields__:
        cp = dataclasses.replace(cp, needs_layout_passes=False)

    @functools.partial(
        pl.kernel,
        compiler_params=cp,
        out_type=jax.ShapeDtypeStruct((_B, _N_OUT, _C), jnp.float32),
        mesh=mesh,
        scratch_types=[
            pltpu.VMEM((_EPT,), jnp.int32),
            pltpu.VMEM((_NCH, _K), jnp.int32),
            pltpu.VMEM((_EPT,), jnp.bfloat16),
            pltpu.VMEM((_K, _C), jnp.bfloat16),
            pltpu.VMEM((_K, _C), jnp.bfloat16),
            pltpu.VMEM((_K, _C), jnp.float32),
            pltpu.VMEM((_K, _C), jnp.float32),
            pltpu.VMEM_SHARED((_N_OUT, _C), jnp.float32),
            pltpu.SemaphoreType.DMA,
            pltpu.SemaphoreType.DMA,
            pltpu.SemaphoreType.DMA,
            pltpu.SemaphoreType.DMA,
        ],
    )
    def k(y_hbm, rows_hbm, cols_hbm, vals_hbm, bias_hbm, out_hbm,
          colsb_v, rows_v, vals_v, g0, g1, s0, s1, acc, gs0, gs1, ss0, ss1):
        gbufs = (g0, g1)
        sbufs = (s0, s1)
        gsems = (gs0, gs1)
        ssems = (ss0, ss1)
        cid = lax.axis_index("c")
        sid = lax.axis_index("s")
        ebase = sid * _EPT
        rbase = sid * _RPT

        def gather_start(b, j, u):
            pltpu.async_copy(
                y_hbm.at[b].at[colsb_v.at[pl.ds(j * _K, _K)]],
                gbufs[u], gsems[u])

        def gather_wait(b, j, u):
            pltpu.make_async_copy(
                y_hbm.at[b].at[colsb_v.at[pl.ds(j * _K, _K)]],
                gbufs[u], gsems[u]
            ).wait()

        def scatter_start(j, u):
            pltpu.async_copy(sbufs[u], acc.at[rows_v.at[j]], ssems[u],
                             add=True)

        def scatter_wait(j, u):
            pltpu.make_async_copy(sbufs[u], acc.at[rows_v.at[j]], ssems[u]
                                  ).wait()

        def scale(g, s, j):
            @plsc.parallel_loop(0, _K, step=32, unroll=2)
            def _(k0):
                vpk = vals_v[pl.ds(j * _K + k0, 32)]
                vv = plsc.unpack(vpk, format=plsc.PackFormat.INTERLEAVED,
                                 preferred_element_type=jnp.float32)
                dnums = lax.GatherDimensionNumbers(
                    offset_dims=(), collapsed_slice_dims=(0,),
                    start_index_map=(0,))
                bcasts = [
                    lax.gather(
                        vv[kk % 2], jnp.full((16, 1), kk // 2, jnp.int32),
                        dnums, slice_sizes=(1,),
                        mode=lax.GatherScatterMode.PROMISE_IN_BOUNDS)
                    for kk in range(32)
                ]
                ncs = _C // 32
                for kk in range(32):
                    packs = [g[k0 + kk, pl.ds(c * 32, 32)] for c in range(ncs)]
                    halves = [
                        plsc.unpack(p, format=plsc.PackFormat.INTERLEAVED,
                                    preferred_element_type=jnp.float32)
                        for p in packs
                    ]
                    prods = [(a * bcasts[kk], b * bcasts[kk])
                             for (a, b) in halves]
                    for c in range(ncs):
                        s[k0 + kk, pl.ds(c * 32, 16)] = prods[c][0]
                        s[k0 + kk, pl.ds(c * 32 + 16, 16)] = prods[c][1]

        pltpu.sync_copy(rows_hbm.at[sid], rows_v)
        pltpu.sync_copy(cols_hbm.at[pl.ds(ebase, _EPT)], colsb_v)
        pltpu.sync_copy(vals_hbm.at[pl.ds(ebase, _EPT)], vals_v)

        @pl.loop(0, _BPC)
        def _(bl):
            b = cid * _BPC + bl
            pltpu.sync_copy(bias_hbm.at[pl.ds(rbase, _RPT)],
                            acc.at[pl.ds(rbase, _RPT)])
            plsc.subcore_barrier()

            gather_start(b, 0, 0)
            gather_start(b, 1, 1)

            @pl.loop(0, _NCH, step=2)
            def _(j0):
                for u in range(2):
                    j = j0 + u
                    gather_wait(b, j, u)

                    @pl.when(j >= 2)
                    def _():
                        scatter_wait(j, u)

                    scale(gbufs[u], sbufs[u], j)
                    jn = j + 2

                    @pl.when(jn < _NCH)
                    def _():
                        gather_start(b, jn, u)

                    scatter_start(j, u)

            for u in range(2):
                scatter_wait(_NCH - 2 + u, u)
            plsc.subcore_barrier()
            pltpu.sync_copy(acc.at[pl.ds(rbase, _RPT)],
                            out_hbm.at[b, pl.ds(rbase, _RPT)])

    return k(y_flat, rows3, cols, vals, bias)


def kernel(x, rows, cols, vals, W, bias):
    B, n_in, C = x.shape
    assert (B, n_in, C) == (_B, _N_IN, _C) and rows.shape == (_NNZ,)
    y = _project(x.reshape(B * n_in, C), W[:, _interleave_perm()]
                 ).reshape(B, n_in, C)
    rows3 = rows.reshape(_NS, _NCH, _K)
    return _sc_aggregate(y, rows3, cols, vals.astype(jnp.bfloat16), bias)
